# baseline (speedup 1.0000x reference)
import functools

import jax
import jax.numpy as jnp
from jax import lax
from jax.experimental import pallas as pl
from jax.experimental.pallas import tpu as pltpu
from jax.experimental.pallas import tpu_sc as plsc

NUM_EMBEDDINGS = 1000000
EMBED_DIM = 64
BATCH = 4096
SEQ = 200

TOTAL = BATCH * SEQ
G = 128
NROWS = TOTAL // G
NW = 32
ROWS_PER_W = NROWS // NW
K = 8
CHUNKS = ROWS_PER_W // K


def _make_kernel():
  mesh = plsc.VectorSubcoreMesh(core_axis_name="c", subcore_axis_name="s")

  @functools.partial(
      pl.kernel,
      mesh=mesh,
      compiler_params=pltpu.CompilerParams(use_tc_tiling_on_sc=False),
      out_type=jax.ShapeDtypeStruct((NROWS, G, EMBED_DIM), jnp.float32),
      scratch_types=[
          pltpu.VMEM((K, G), jnp.int32),
          pltpu.VMEM((K, G, EMBED_DIM), jnp.float32),
          pltpu.SemaphoreType.DMA,
      ],
  )
  def gather_kernel(idx_hbm, table_hbm, out_hbm, idx_v, rows_v, sem):
    wid = lax.axis_index("s") * 2 + lax.axis_index("c")
    base = wid * ROWS_PER_W

    def chunk_body(c, _):
      row0 = base + c * K
      pltpu.sync_copy(idx_hbm.at[pl.ds(row0, K)], idx_v)
      copies = []
      for j in range(K):
        copies.append(
            pltpu.async_copy(table_hbm.at[idx_v.at[j]], rows_v.at[j], sem))
      for cp in copies:
        cp.wait()
      pltpu.sync_copy(rows_v, out_hbm.at[pl.ds(row0, K)])
      return 0

    lax.fori_loop(0, CHUNKS, chunk_body, 0)

  return gather_kernel


_gather = _make_kernel()


@jax.jit
def kernel(indices, table):
  idx2d = indices.reshape(NROWS, G).astype(jnp.int32)
  out = _gather(idx2d, table)
  return out.reshape(BATCH, SEQ, EMBED_DIM)


# speedup vs baseline: 1.0092x; 1.0092x over previous
import functools

import jax
import jax.numpy as jnp
from jax import lax
from jax.experimental import pallas as pl
from jax.experimental.pallas import tpu as pltpu
from jax.experimental.pallas import tpu_sc as plsc

NUM_EMBEDDINGS = 1000000
EMBED_DIM = 64
BATCH = 4096
SEQ = 200

TOTAL = BATCH * SEQ
G = 128
NROWS = TOTAL // G
NW = 32
ROWS_PER_W = NROWS // NW
K = 5
CHUNKS = ROWS_PER_W // K
NB = 2


def _make_kernel():
  mesh = plsc.VectorSubcoreMesh(core_axis_name="c", subcore_axis_name="s")

  @functools.partial(
      pl.kernel,
      mesh=mesh,
      compiler_params=pltpu.CompilerParams(use_tc_tiling_on_sc=False),
      out_type=jax.ShapeDtypeStruct((NROWS, G, EMBED_DIM), jnp.float32),
      scratch_types=[
          pltpu.VMEM((NB, K, G), jnp.int32),
          pltpu.VMEM((NB, K, G, EMBED_DIM), jnp.float32),
          pltpu.SemaphoreType.DMA((NB,)),
          pltpu.SemaphoreType.DMA((NB,)),
      ],
  )
  def gather_kernel(idx_hbm, table_hbm, out_hbm, idx_v, rows_v, gsem, wsem):
    wid = lax.axis_index("s") * 2 + lax.axis_index("c")
    base = wid * ROWS_PER_W

    def fire(c, b):
      row0 = base + c * K
      pltpu.sync_copy(idx_hbm.at[pl.ds(row0, K)], idx_v.at[b])
      for j in range(K):
        pltpu.async_copy(table_hbm.at[idx_v.at[b, j]], rows_v.at[b, j],
                         gsem.at[b])

    def drain_writeback(c, b):
      row0 = base + c * K
      for j in range(K):
        pltpu.make_async_copy(table_hbm.at[idx_v.at[b, j]], rows_v.at[b, j],
                              gsem.at[b]).wait()
      pltpu.async_copy(rows_v.at[b], out_hbm.at[pl.ds(row0, K)], wsem.at[b])

    def wait_writeback(c, b):
      row0 = base + c * K
      pltpu.make_async_copy(rows_v.at[b], out_hbm.at[pl.ds(row0, K)],
                            wsem.at[b]).wait()

    fire(0, 0)

    def body(t, _):
      c0 = t * NB
      fire(c0 + 1, 1)
      drain_writeback(c0, 0)
      @pl.when(t + 1 < CHUNKS // NB)
      def _():
        wait_writeback(c0, 0)
        fire(c0 + 2, 0)
      drain_writeback(c0 + 1, 1)
      @pl.when(t + 1 < CHUNKS // NB)
      def _():
        wait_writeback(c0 + 1, 1)
      return 0

    lax.fori_loop(0, CHUNKS // NB, body, 0)
    wait_writeback(CHUNKS - 2, 0)
    wait_writeback(CHUNKS - 1, 1)

  return gather_kernel


_gather = _make_kernel()


@jax.jit
def kernel(indices, table):
  idx2d = indices.reshape(NROWS, G).astype(jnp.int32)
  out = _gather(idx2d, table)
  return out.reshape(BATCH, SEQ, EMBED_DIM)
